# baseline (speedup 1.0000x reference)
import functools

import jax
import jax.numpy as jnp
from jax.experimental import pallas as pl
from jax.experimental.pallas import tpu as pltpu
from jax.experimental.pallas import tpu_sc as plsc

_E = 64
_TOPK = 2
_H = 768
_I = 768
_T = 2048
_R = _T * _TOPK
_BM = 128
_MAXPAD = _R + _E * _BM
_NBLK = _MAXPAD // _BM
_LA = 3
_NRING = 5
_RXN = _NBLK + _LA + 1

_INTERPRET = False

_NC = 2
_NS = 16
_NW = _NC * _NS
_GCH = 64
_CCH = 32


def _sc_gather_body(idx_hbm, tab_hbm, out_hbm, idx_v, buf, sem):
    wid = jax.lax.axis_index("s") * _NC + jax.lax.axis_index("c")
    rpw = _MAXPAD // _NW
    base = wid * rpw

    def body(ci, _):
        off = base + ci * _GCH
        pltpu.sync_copy(idx_hbm.at[pl.ds(off, _GCH)], idx_v)
        pltpu.async_copy(tab_hbm.at[idx_v], buf, sem).wait()
        pltpu.sync_copy(buf, out_hbm.at[pl.ds(off, _GCH)])
        return 0

    jax.lax.fori_loop(0, rpw // _GCH, body, 0, unroll=False)


_sc_gather = functools.partial(
    pl.kernel,
    out_type=jax.ShapeDtypeStruct((_MAXPAD, _H), jnp.float32),
    mesh=plsc.VectorSubcoreMesh(core_axis_name="c", subcore_axis_name="s"),
    scratch_types=[
        pltpu.VMEM((_GCH,), jnp.int32),
        pltpu.VMEM((_GCH, _H), jnp.float32),
        pltpu.SemaphoreType.DMA,
    ],
)(_sc_gather_body)


def _sc_combine_body(p0_hbm, p1_hbm, y_hbm, out_hbm, i0, i1, b0, b1v,
                     sem0, sem1):
    wid = jax.lax.axis_index("s") * _NC + jax.lax.axis_index("c")
    tpw = _T // _NW
    base = wid * tpw

    def chunk(ci, _):
        off = base + ci * _CCH
        pltpu.sync_copy(p0_hbm.at[pl.ds(off, _CCH)], i0)
        pltpu.sync_copy(p1_hbm.at[pl.ds(off, _CCH)], i1)
        c0 = pltpu.async_copy(y_hbm.at[i0], b0, sem0)
        c1 = pltpu.async_copy(y_hbm.at[i1], b1v, sem1)
        c0.wait()
        c1.wait()

        def row(r, _2):
            for v in range(_H // 16):
                sl = pl.ds(v * 16, 16)
                b0[r, sl] = b0[r, sl] + b1v[r, sl]
            return 0

        jax.lax.fori_loop(0, _CCH, row, 0, unroll=False)
        pltpu.sync_copy(b0, out_hbm.at[pl.ds(off, _CCH)])
        return 0

    jax.lax.fori_loop(0, tpw // _CCH, chunk, 0, unroll=False)


_sc_combine = functools.partial(
    pl.kernel,
    out_type=jax.ShapeDtypeStruct((_T, _H), jnp.float32),
    mesh=plsc.VectorSubcoreMesh(core_axis_name="c", subcore_axis_name="s"),
    scratch_types=[
        pltpu.VMEM((_CCH,), jnp.int32),
        pltpu.VMEM((_CCH,), jnp.int32),
        pltpu.VMEM((_CCH, _H), jnp.float32),
        pltpu.VMEM((_CCH, _H), jnp.float32),
        pltpu.SemaphoreType.DMA,
        pltpu.SemaphoreType.DMA,
    ],
)(_sc_combine_body)


def _ffn_block(info_ref, runid_ref, fb_ref, runx_ref,
               x_ref, w1_hbm, b1_ref, sel_ref, w2_hbm, b2_ref, wp_ref,
               out_ref, w1bufa, w1bufb, w2bufa, w2bufb,
               sem1a, sem1b, sem2a, sem2b):
    i = pl.program_id(0)
    nused = info_ref[_NBLK]
    nruns = runx_ref[_RXN - 1]

    def copies(run, slot):
        e = runx_ref[run]
        return (
            pltpu.make_async_copy(w1_hbm.at[e, pl.ds(0, _I)],
                                  w1bufa.at[slot], sem1a.at[slot]),
            pltpu.make_async_copy(w1_hbm.at[e, pl.ds(_I, _I)],
                                  w1bufb.at[slot], sem1b.at[slot]),
            pltpu.make_async_copy(w2_hbm.at[e, pl.ds(0, _H // 2)],
                                  w2bufa.at[slot], sem2a.at[slot]),
            pltpu.make_async_copy(w2_hbm.at[e, pl.ds(_H // 2, _H // 2)],
                                  w2bufb.at[slot], sem2b.at[slot]),
        )

    @pl.when(i == 0)
    def _():
        for k in range(_LA):
            @pl.when(k < nruns)
            def _():
                for c in copies(k, k):
                    c.start()

    @pl.when((fb_ref[i] == 1) & (i < nused))
    def _():
        r = runid_ref[i]

        @pl.when(r + _LA < nruns)
        def _():
            for c in copies(r + _LA, jax.lax.rem(r + _LA, _NRING)):
                c.start()

        for c in copies(r, jax.lax.rem(r, _NRING)):
            c.wait()

    @pl.when(i < nused)
    def _():
        slot = jax.lax.rem(runid_ref[i], _NRING)
        x = x_ref[...]
        b1 = b1_ref[0, 0]

        def shalf(wbuf, k):
            h = jax.lax.dot_general(x, wbuf[slot], (((1,), (1,)), ((), ())),
                                    preferred_element_type=jnp.float32)
            h = h + jax.lax.slice_in_dim(b1, k * _I, (k + 1) * _I, axis=0)
            hr = pltpu.roll(h, _I - 1, 1)
            p = h * jax.nn.sigmoid(1.702 * h) * (hr + 1.0)
            return jax.lax.dot_general(p, sel_ref[...], (((1,), (0,)), ((), ())),
                                       preferred_element_type=jnp.float32)

        s = jnp.concatenate([shalf(w1bufa, 0), shalf(w1bufb, 1)], axis=1)
        ya = jax.lax.dot_general(s, w2bufa[slot], (((1,), (1,)), ((), ())),
                                 preferred_element_type=jnp.float32)
        yb = jax.lax.dot_general(s, w2bufb[slot], (((1,), (1,)), ((), ())),
                                 preferred_element_type=jnp.float32)
        y = jnp.concatenate([ya, yb], axis=1) + b2_ref[0, 0]
        out_ref[...] = y * wp_ref[0, 0][:, None]


def kernel(hidden_states, gate_w, gate_b, w1, b1, w2, b2):
    bsz, seq, hd = hidden_states.shape
    x2 = hidden_states.reshape(-1, hd)

    logits = x2 @ gate_w.T + gate_b
    probs = jax.nn.softmax(logits, axis=-1)
    vals, idx = jax.lax.top_k(probs, _TOPK)
    vals = vals / jnp.sum(vals, axis=-1, keepdims=True)

    e_flat = idx.reshape(-1).astype(jnp.int32)
    v_flat = vals.reshape(-1)
    order = jnp.argsort(e_flat, stable=True)
    rank = jnp.zeros((_R,), jnp.int32).at[order].set(
        jnp.arange(_R, dtype=jnp.int32))
    counts = jnp.bincount(e_flat, length=_E).astype(jnp.int32)
    pcounts = ((counts + _BM - 1) // _BM) * _BM
    pc_cum = jnp.cumsum(pcounts).astype(jnp.int32)
    pstart = pc_cum - pcounts
    g_cum = jnp.cumsum(counts).astype(jnp.int32)
    gstart = g_cum - counts
    total_pad = pc_cum[-1]
    nused = (total_pad // _BM).astype(jnp.int32)
    pos = pstart[e_flat] + (rank - gstart[e_flat])
    src_tok = jnp.zeros((_MAXPAD,), jnp.int32).at[pos].set(
        jnp.arange(_R, dtype=jnp.int32) // _TOPK)
    w_pad = jnp.zeros((_MAXPAD,), jnp.float32).at[pos].set(v_flat)
    queries = (jnp.arange(_NBLK, dtype=jnp.int32) * _BM).astype(jnp.int32)
    be = jnp.searchsorted(pc_cum, queries, side="right").astype(jnp.int32)
    be_last = be[jnp.maximum(nused - 1, 0)]
    be = jnp.where(queries < total_pad, be, be_last)
    info = jnp.concatenate([be, nused[None]])
    fb = jnp.concatenate([jnp.ones((1,), jnp.int32),
                          (be[1:] != be[:-1]).astype(jnp.int32)])
    fb = fb * (queries < total_pad).astype(jnp.int32)
    runid = jnp.cumsum(fb).astype(jnp.int32) - 1
    nruns = jnp.sum(fb).astype(jnp.int32)
    runx = jnp.zeros((_RXN,), jnp.int32).at[runid].set(be)
    runx = runx.at[_RXN - 1].set(nruns)

    x_pad = _sc_gather(src_tok, x2)

    b1r = b1.reshape(_E, 1, 2 * _I)
    b2r = b2.reshape(_E, 1, _H)
    wpr = w_pad.reshape(_NBLK, 1, _BM)
    sel = (jnp.arange(_I, dtype=jnp.int32)[:, None]
           == 2 * jnp.arange(_I // 2, dtype=jnp.int32)[None, :]
           ).astype(jnp.float32)
    grid_spec = pltpu.PrefetchScalarGridSpec(
        num_scalar_prefetch=4,
        grid=(_NBLK,),
        in_specs=[
            pl.BlockSpec((_BM, _H),
                         lambda i, *s: (jnp.minimum(i, s[0][_NBLK] - 1), 0)),
            pl.BlockSpec(memory_space=pltpu.MemorySpace.HBM),
            pl.BlockSpec((1, 1, 2 * _I), lambda i, *s: (s[0][i], 0, 0)),
            pl.BlockSpec((_I, _I // 2), lambda i, *s: (0, 0)),
            pl.BlockSpec(memory_space=pltpu.MemorySpace.HBM),
            pl.BlockSpec((1, 1, _H), lambda i, *s: (s[0][i], 0, 0)),
            pl.BlockSpec((1, 1, _BM), lambda i, *s: (i, 0, 0)),
        ],
        out_specs=pl.BlockSpec(
            (_BM, _H), lambda i, *s: (jnp.minimum(i, s[0][_NBLK] - 1), 0)),
        scratch_shapes=[
            pltpu.VMEM((_NRING, _I, _H), jnp.float32),
            pltpu.VMEM((_NRING, _I, _H), jnp.float32),
            pltpu.VMEM((_NRING, _H // 2, _I), jnp.float32),
            pltpu.VMEM((_NRING, _H // 2, _I), jnp.float32),
            pltpu.SemaphoreType.DMA((_NRING,)),
            pltpu.SemaphoreType.DMA((_NRING,)),
            pltpu.SemaphoreType.DMA((_NRING,)),
            pltpu.SemaphoreType.DMA((_NRING,)),
        ],
    )
    y_pad = pl.pallas_call(
        _ffn_block,
        grid_spec=grid_spec,
        out_shape=jax.ShapeDtypeStruct((_MAXPAD, _H), jnp.float32),
        compiler_params=pltpu.CompilerParams(
            dimension_semantics=("arbitrary",)),
        interpret=_INTERPRET,
    )(info, runid, fb, runx, x_pad, w1, b1r, sel, w2, b2r, wpr)

    p2 = pos.reshape(_T, _TOPK)
    out2 = _sc_combine(p2[:, 0], p2[:, 1], y_pad)
    return out2.reshape(bsz, seq, hd)


# speedup vs baseline: 1.8948x; 1.8948x over previous
import functools

import jax
import jax.numpy as jnp
from jax.experimental import pallas as pl
from jax.experimental.pallas import tpu as pltpu
from jax.experimental.pallas import tpu_sc as plsc

_E = 64
_TOPK = 2
_H = 768
_I = 768
_T = 2048
_R = _T * _TOPK
_BM = 128
_MAXPAD = _R + _E * _BM
_NBLK = _MAXPAD // _BM
_LA = 3
_NRING = 5
_RXN = _NBLK + _LA + 1

_INTERPRET = False

_NC = 2
_NS = 16
_NW = _NC * _NS
_GCH = 128
_CCH = 32


def _sc_gather_body(idx_hbm, tab_hbm, out_hbm, idx_v, buf, sem):
    wid = jax.lax.axis_index("s") * _NC + jax.lax.axis_index("c")
    rpw = _MAXPAD // _NW
    base = wid * rpw

    def body(ci, _):
        off = base + ci * _GCH
        pltpu.sync_copy(idx_hbm.at[pl.ds(off, _GCH)], idx_v)
        pltpu.async_copy(tab_hbm.at[idx_v], buf, sem).wait()
        pltpu.sync_copy(buf, out_hbm.at[pl.ds(off, _GCH)])
        return 0

    jax.lax.fori_loop(0, rpw // _GCH, body, 0, unroll=False)


_sc_gather = functools.partial(
    pl.kernel,
    out_type=jax.ShapeDtypeStruct((_MAXPAD, _H), jnp.float32),
    mesh=plsc.VectorSubcoreMesh(core_axis_name="c", subcore_axis_name="s"),
    scratch_types=[
        pltpu.VMEM((_GCH,), jnp.int32),
        pltpu.VMEM((_GCH, _H), jnp.float32),
        pltpu.SemaphoreType.DMA,
    ],
)(_sc_gather_body)


def _sc_combine_body(p0_hbm, p1_hbm, y_hbm, out_hbm, i0, i1, b0, b1v,
                     sem0, sem1):
    wid = jax.lax.axis_index("s") * _NC + jax.lax.axis_index("c")
    tpw = _T // _NW
    base = wid * tpw

    def chunk(ci, _):
        off = base + ci * _CCH
        pltpu.sync_copy(p0_hbm.at[pl.ds(off, _CCH)], i0)
        pltpu.sync_copy(p1_hbm.at[pl.ds(off, _CCH)], i1)
        c0 = pltpu.async_copy(y_hbm.at[i0], b0, sem0)
        c1 = pltpu.async_copy(y_hbm.at[i1], b1v, sem1)
        c0.wait()
        c1.wait()

        def row(r, _2):
            for v in range(_H // 16):
                sl = pl.ds(v * 16, 16)
                b0[r, sl] = b0[r, sl] + b1v[r, sl]
            return 0

        jax.lax.fori_loop(0, _CCH, row, 0, unroll=False)
        pltpu.sync_copy(b0, out_hbm.at[pl.ds(off, _CCH)])
        return 0

    jax.lax.fori_loop(0, tpw // _CCH, chunk, 0, unroll=False)


_sc_combine = functools.partial(
    pl.kernel,
    out_type=jax.ShapeDtypeStruct((_T, _H), jnp.float32),
    mesh=plsc.VectorSubcoreMesh(core_axis_name="c", subcore_axis_name="s"),
    scratch_types=[
        pltpu.VMEM((_CCH,), jnp.int32),
        pltpu.VMEM((_CCH,), jnp.int32),
        pltpu.VMEM((_CCH, _H), jnp.float32),
        pltpu.VMEM((_CCH, _H), jnp.float32),
        pltpu.SemaphoreType.DMA,
        pltpu.SemaphoreType.DMA,
    ],
)(_sc_combine_body)


def _ffn_block(info_ref, runid_ref, fb_ref, runx_ref,
               x_ref, w1_hbm, b1_ref, sel_ref, w2_hbm, b2_ref, wp_ref,
               out_ref, w1bufa, w1bufb, w2bufa, w2bufb,
               sem1a, sem1b, sem2a, sem2b):
    i = pl.program_id(0)
    nused = info_ref[_NBLK]
    nruns = runx_ref[_RXN - 1]

    def copies(run, slot):
        e = runx_ref[run]
        return (
            pltpu.make_async_copy(w1_hbm.at[e, pl.ds(0, _I)],
                                  w1bufa.at[slot], sem1a.at[slot]),
            pltpu.make_async_copy(w1_hbm.at[e, pl.ds(_I, _I)],
                                  w1bufb.at[slot], sem1b.at[slot]),
            pltpu.make_async_copy(w2_hbm.at[e, pl.ds(0, _H // 2)],
                                  w2bufa.at[slot], sem2a.at[slot]),
            pltpu.make_async_copy(w2_hbm.at[e, pl.ds(_H // 2, _H // 2)],
                                  w2bufb.at[slot], sem2b.at[slot]),
        )

    @pl.when(i == 0)
    def _():
        for k in range(_LA):
            @pl.when(k < nruns)
            def _():
                for c in copies(k, k):
                    c.start()

    @pl.when((fb_ref[i] == 1) & (i < nused))
    def _():
        r = runid_ref[i]

        @pl.when(r + _LA < nruns)
        def _():
            for c in copies(r + _LA, jax.lax.rem(r + _LA, _NRING)):
                c.start()

        for c in copies(r, jax.lax.rem(r, _NRING)):
            c.wait()

    @pl.when(i < nused)
    def _():
        slot = jax.lax.rem(runid_ref[i], _NRING)
        x = x_ref[...]
        b1 = b1_ref[0, 0]

        def shalf(wbuf, k):
            h = jax.lax.dot_general(x, wbuf[slot], (((1,), (1,)), ((), ())),
                                    preferred_element_type=jnp.float32)
            h = h + jax.lax.slice_in_dim(b1, k * _I, (k + 1) * _I, axis=0)
            hr = pltpu.roll(h, _I - 1, 1)
            p = h * jax.nn.sigmoid(1.702 * h) * (hr + 1.0)
            return jax.lax.dot_general(p, sel_ref[...], (((1,), (0,)), ((), ())),
                                       preferred_element_type=jnp.float32)

        s = jnp.concatenate([shalf(w1bufa, 0), shalf(w1bufb, 1)], axis=1)
        ya = jax.lax.dot_general(s, w2bufa[slot], (((1,), (1,)), ((), ())),
                                 preferred_element_type=jnp.float32)
        yb = jax.lax.dot_general(s, w2bufb[slot], (((1,), (1,)), ((), ())),
                                 preferred_element_type=jnp.float32)
        y = jnp.concatenate([ya, yb], axis=1) + b2_ref[0, 0]
        out_ref[...] = y * wp_ref[0, 0][:, None]


def kernel(hidden_states, gate_w, gate_b, w1, b1, w2, b2):
    bsz, seq, hd = hidden_states.shape
    x2 = hidden_states.reshape(-1, hd)

    logits = x2 @ gate_w.T + gate_b
    probs = jax.nn.softmax(logits, axis=-1)
    vals, idx = jax.lax.top_k(probs, _TOPK)
    vals = vals / jnp.sum(vals, axis=-1, keepdims=True)

    e_flat = idx.reshape(-1).astype(jnp.int32)
    v_flat = vals.reshape(-1)
    order = jnp.argsort(e_flat, stable=True)
    rank = jnp.zeros((_R,), jnp.int32).at[order].set(
        jnp.arange(_R, dtype=jnp.int32))
    counts = jnp.bincount(e_flat, length=_E).astype(jnp.int32)
    pcounts = ((counts + _BM - 1) // _BM) * _BM
    pc_cum = jnp.cumsum(pcounts).astype(jnp.int32)
    pstart = pc_cum - pcounts
    g_cum = jnp.cumsum(counts).astype(jnp.int32)
    gstart = g_cum - counts
    total_pad = pc_cum[-1]
    nused = (total_pad // _BM).astype(jnp.int32)
    pos = pstart[e_flat] + (rank - gstart[e_flat])
    src_tok = (jnp.arange(_MAXPAD, dtype=jnp.int32) % _T).at[pos].set(
        jnp.arange(_R, dtype=jnp.int32) // _TOPK)
    w_pad = jnp.zeros((_MAXPAD,), jnp.float32).at[pos].set(v_flat)
    queries = (jnp.arange(_NBLK, dtype=jnp.int32) * _BM).astype(jnp.int32)
    be = jnp.searchsorted(pc_cum, queries, side="right").astype(jnp.int32)
    be_last = be[jnp.maximum(nused - 1, 0)]
    be = jnp.where(queries < total_pad, be, be_last)
    info = jnp.concatenate([be, nused[None]])
    fb = jnp.concatenate([jnp.ones((1,), jnp.int32),
                          (be[1:] != be[:-1]).astype(jnp.int32)])
    fb = fb * (queries < total_pad).astype(jnp.int32)
    runid = jnp.cumsum(fb).astype(jnp.int32) - 1
    nruns = jnp.sum(fb).astype(jnp.int32)
    runx = jnp.zeros((_RXN,), jnp.int32).at[runid].set(be)
    runx = runx.at[_RXN - 1].set(nruns)

    x_pad = _sc_gather(src_tok, x2)

    b1r = b1.reshape(_E, 1, 2 * _I)
    b2r = b2.reshape(_E, 1, _H)
    wpr = w_pad.reshape(_NBLK, 1, _BM)
    sel = (jnp.arange(_I, dtype=jnp.int32)[:, None]
           == 2 * jnp.arange(_I // 2, dtype=jnp.int32)[None, :]
           ).astype(jnp.float32)
    grid_spec = pltpu.PrefetchScalarGridSpec(
        num_scalar_prefetch=4,
        grid=(_NBLK,),
        in_specs=[
            pl.BlockSpec((_BM, _H),
                         lambda i, *s: (jnp.minimum(i, s[0][_NBLK] - 1), 0)),
            pl.BlockSpec(memory_space=pltpu.MemorySpace.HBM),
            pl.BlockSpec((1, 1, 2 * _I), lambda i, *s: (s[0][i], 0, 0)),
            pl.BlockSpec((_I, _I // 2), lambda i, *s: (0, 0)),
            pl.BlockSpec(memory_space=pltpu.MemorySpace.HBM),
            pl.BlockSpec((1, 1, _H), lambda i, *s: (s[0][i], 0, 0)),
            pl.BlockSpec((1, 1, _BM), lambda i, *s: (i, 0, 0)),
        ],
        out_specs=pl.BlockSpec(
            (_BM, _H), lambda i, *s: (jnp.minimum(i, s[0][_NBLK] - 1), 0)),
        scratch_shapes=[
            pltpu.VMEM((_NRING, _I, _H), jnp.float32),
            pltpu.VMEM((_NRING, _I, _H), jnp.float32),
            pltpu.VMEM((_NRING, _H // 2, _I), jnp.float32),
            pltpu.VMEM((_NRING, _H // 2, _I), jnp.float32),
            pltpu.SemaphoreType.DMA((_NRING,)),
            pltpu.SemaphoreType.DMA((_NRING,)),
            pltpu.SemaphoreType.DMA((_NRING,)),
            pltpu.SemaphoreType.DMA((_NRING,)),
        ],
    )
    y_pad = pl.pallas_call(
        _ffn_block,
        grid_spec=grid_spec,
        out_shape=jax.ShapeDtypeStruct((_MAXPAD, _H), jnp.float32),
        compiler_params=pltpu.CompilerParams(
            dimension_semantics=("arbitrary",)),
        interpret=_INTERPRET,
    )(info, runid, fb, runx, x_pad, w1, b1r, sel, w2, b2r, wpr)

    p2 = pos.reshape(_T, _TOPK)
    out2 = _sc_combine(p2[:, 0], p2[:, 1], y_pad)
    return out2.reshape(bsz, seq, hd)


# speedup vs baseline: 2.1649x; 1.1425x over previous
import functools

import jax
import jax.numpy as jnp
from jax.experimental import pallas as pl
from jax.experimental.pallas import tpu as pltpu
from jax.experimental.pallas import tpu_sc as plsc

_E = 64
_TOPK = 2
_H = 768
_I = 768
_T = 2048
_R = _T * _TOPK
_BM = 128
_MAXPAD = _R + _E * _BM
_NBLK = _MAXPAD // _BM
_LA = 3
_NRING = 5
_RXN = _NBLK + _LA + 1

_NC = 2
_NS = 16
_NW = _NC * _NS
_GCH = 128
_CCH = 32


def _sc_gather_body(idx_hbm, tab_hbm, out_hbm, idx_v, buf, sem):
    wid = jax.lax.axis_index("s") * _NC + jax.lax.axis_index("c")
    rpw = _MAXPAD // _NW
    base = wid * rpw

    def body(ci, _):
        off = base + ci * _GCH
        pltpu.sync_copy(idx_hbm.at[pl.ds(off, _GCH)], idx_v)
        pltpu.async_copy(tab_hbm.at[idx_v], buf, sem).wait()
        pltpu.sync_copy(buf, out_hbm.at[pl.ds(off, _GCH)])
        return 0

    jax.lax.fori_loop(0, rpw // _GCH, body, 0, unroll=False)


_sc_gather = functools.partial(
    pl.kernel,
    out_type=jax.ShapeDtypeStruct((_MAXPAD, _H), jnp.float32),
    mesh=plsc.VectorSubcoreMesh(core_axis_name="c", subcore_axis_name="s"),
    scratch_types=[
        pltpu.VMEM((_GCH,), jnp.int32),
        pltpu.VMEM((_GCH, _H), jnp.float32),
        pltpu.SemaphoreType.DMA,
    ],
)(_sc_gather_body)


def _sc_combine_body(p0_hbm, p1_hbm, y_hbm, out_hbm, i0, i1, b0, b1v,
                     sem0, sem1):
    wid = jax.lax.axis_index("s") * _NC + jax.lax.axis_index("c")
    tpw = _T // _NW
    base = wid * tpw

    def chunk(ci, _):
        off = base + ci * _CCH
        pltpu.sync_copy(p0_hbm.at[pl.ds(off, _CCH)], i0)
        pltpu.sync_copy(p1_hbm.at[pl.ds(off, _CCH)], i1)
        c0 = pltpu.async_copy(y_hbm.at[i0], b0, sem0)
        c1 = pltpu.async_copy(y_hbm.at[i1], b1v, sem1)
        c0.wait()
        c1.wait()

        def row(r, _2):
            for v in range(_H // 16):
                sl = pl.ds(v * 16, 16)
                b0[r, sl] = b0[r, sl] + b1v[r, sl]
            return 0

        jax.lax.fori_loop(0, _CCH, row, 0, unroll=False)
        pltpu.sync_copy(b0, out_hbm.at[pl.ds(off, _CCH)])
        return 0

    jax.lax.fori_loop(0, tpw // _CCH, chunk, 0, unroll=False)


_sc_combine = functools.partial(
    pl.kernel,
    out_type=jax.ShapeDtypeStruct((_T, _H), jnp.float32),
    mesh=plsc.VectorSubcoreMesh(core_axis_name="c", subcore_axis_name="s"),
    scratch_types=[
        pltpu.VMEM((_CCH,), jnp.int32),
        pltpu.VMEM((_CCH,), jnp.int32),
        pltpu.VMEM((_CCH, _H), jnp.float32),
        pltpu.VMEM((_CCH, _H), jnp.float32),
        pltpu.SemaphoreType.DMA,
        pltpu.SemaphoreType.DMA,
    ],
)(_sc_combine_body)


def _router_block(x_ref, gw_ref, gb_ref, out_ref):
    x = x_ref[...]
    logits = jax.lax.dot_general(x, gw_ref[...], (((1,), (1,)), ((), ())),
                                 preferred_element_type=jnp.float32)
    logits = logits + gb_ref[0]
    m = jnp.max(logits, axis=1, keepdims=True)
    p = jnp.exp(logits - m)
    probs = p / jnp.sum(p, axis=1, keepdims=True)
    ii = jax.lax.broadcasted_iota(jnp.int32, probs.shape, 1)
    m1 = jnp.max(probs, axis=1, keepdims=True)
    i1 = jnp.min(jnp.where(probs == m1, ii, _E), axis=1, keepdims=True)
    masked = jnp.where(ii == i1, -1.0, probs)
    m2 = jnp.max(masked, axis=1, keepdims=True)
    i2 = jnp.min(jnp.where((masked == m2) & (ii != i1), ii, _E),
                 axis=1, keepdims=True)
    tot = m1 + m2
    out_ref[...] = jnp.concatenate(
        [m1 / tot, m2 / tot, i1.astype(jnp.float32), i2.astype(jnp.float32),
         jnp.zeros((_T, 4), jnp.float32)], axis=1)


def _ffn_block(info_ref, runid_ref, fb_ref, runx_ref,
               x_ref, w1_hbm, b1_ref, sel_ref, w2_hbm, b2_ref, wp_ref,
               out_ref, w1bufa, w1bufb, w2bufa, w2bufb,
               sem1a, sem1b, sem2a, sem2b):
    i = pl.program_id(0)
    nused = info_ref[_NBLK]
    nruns = runx_ref[_RXN - 1]

    def copies(run, slot):
        e = runx_ref[run]
        return (
            pltpu.make_async_copy(w1_hbm.at[e, pl.ds(0, _I)],
                                  w1bufa.at[slot], sem1a.at[slot]),
            pltpu.make_async_copy(w1_hbm.at[e, pl.ds(_I, _I)],
                                  w1bufb.at[slot], sem1b.at[slot]),
            pltpu.make_async_copy(w2_hbm.at[e, pl.ds(0, _H // 2)],
                                  w2bufa.at[slot], sem2a.at[slot]),
            pltpu.make_async_copy(w2_hbm.at[e, pl.ds(_H // 2, _H // 2)],
                                  w2bufb.at[slot], sem2b.at[slot]),
        )

    @pl.when(i == 0)
    def _():
        for k in range(_LA):
            @pl.when(k < nruns)
            def _():
                for c in copies(k, k):
                    c.start()

    @pl.when((fb_ref[i] == 1) & (i < nused))
    def _():
        r = runid_ref[i]

        @pl.when(r + _LA < nruns)
        def _():
            for c in copies(r + _LA, jax.lax.rem(r + _LA, _NRING)):
                c.start()

        for c in copies(r, jax.lax.rem(r, _NRING)):
            c.wait()

    @pl.when(i < nused)
    def _():
        slot = jax.lax.rem(runid_ref[i], _NRING)
        x = x_ref[...]
        b1 = b1_ref[0, 0]

        def shalf(wbuf, k):
            h = jax.lax.dot_general(x, wbuf[slot], (((1,), (1,)), ((), ())),
                                    preferred_element_type=jnp.float32)
            h = h + jax.lax.slice_in_dim(b1, k * _I, (k + 1) * _I, axis=0)
            hr = pltpu.roll(h, _I - 1, 1)
            p = h * jax.nn.sigmoid(1.702 * h) * (hr + 1.0)
            return jax.lax.dot_general(p, sel_ref[...], (((1,), (0,)), ((), ())),
                                       preferred_element_type=jnp.float32)

        s = jnp.concatenate([shalf(w1bufa, 0), shalf(w1bufb, 1)], axis=1)
        ya = jax.lax.dot_general(s, w2bufa[slot], (((1,), (1,)), ((), ())),
                                 preferred_element_type=jnp.float32)
        yb = jax.lax.dot_general(s, w2bufb[slot], (((1,), (1,)), ((), ())),
                                 preferred_element_type=jnp.float32)
        y = jnp.concatenate([ya, yb], axis=1) + b2_ref[0, 0]
        out_ref[...] = y * wp_ref[0, 0][:, None]


def kernel(hidden_states, gate_w, gate_b, w1, b1, w2, b2):
    bsz, seq, hd = hidden_states.shape
    x2 = hidden_states.reshape(-1, hd)

    rout = pl.pallas_call(
        _router_block,
        grid=(1,),
        in_specs=[
            pl.BlockSpec((_T, _H), lambda i: (0, 0)),
            pl.BlockSpec((_E, _H), lambda i: (0, 0)),
            pl.BlockSpec((1, _E), lambda i: (0, 0)),
        ],
        out_specs=pl.BlockSpec((_T, 8), lambda i: (0, 0)),
        out_shape=jax.ShapeDtypeStruct((_T, 8), jnp.float32),
    )(x2, gate_w, gate_b.reshape(1, _E))
    vals = rout[:, 0:2]
    idx = rout[:, 2:4].astype(jnp.int32)

    e_flat = idx.reshape(-1)
    v_flat = vals.reshape(-1)
    onehot = (e_flat[:, None]
              == jnp.arange(_E, dtype=jnp.int32)[None, :]).astype(jnp.int32)
    cum = jnp.cumsum(onehot, axis=0)
    counts = cum[-1]
    pcounts = ((counts + _BM - 1) // _BM) * _BM
    pc_cum = jnp.cumsum(pcounts).astype(jnp.int32)
    pstart = pc_cum - pcounts
    total_pad = pc_cum[-1]
    nused = (total_pad // _BM).astype(jnp.int32)
    rank_within = jnp.take_along_axis(cum, e_flat[:, None], axis=1)[:, 0] - 1
    pos = pstart[e_flat] + rank_within
    src_tok = (jnp.arange(_MAXPAD, dtype=jnp.int32) % _T).at[pos].set(
        jnp.arange(_R, dtype=jnp.int32) // _TOPK)
    w_pad = jnp.zeros((_MAXPAD,), jnp.float32).at[pos].set(v_flat)
    queries = (jnp.arange(_NBLK, dtype=jnp.int32) * _BM).astype(jnp.int32)
    be = jnp.searchsorted(pc_cum, queries, side="right").astype(jnp.int32)
    be_last = be[jnp.maximum(nused - 1, 0)]
    be = jnp.where(queries < total_pad, be, be_last)
    info = jnp.concatenate([be, nused[None]])
    fb = jnp.concatenate([jnp.ones((1,), jnp.int32),
                          (be[1:] != be[:-1]).astype(jnp.int32)])
    fb = fb * (queries < total_pad).astype(jnp.int32)
    runid = jnp.cumsum(fb).astype(jnp.int32) - 1
    nruns = jnp.sum(fb).astype(jnp.int32)
    runx = jnp.zeros((_RXN,), jnp.int32).at[runid].set(be)
    runx = runx.at[_RXN - 1].set(nruns)

    x_pad = _sc_gather(src_tok, x2)

    b1r = b1.reshape(_E, 1, 2 * _I)
    b2r = b2.reshape(_E, 1, _H)
    wpr = w_pad.reshape(_NBLK, 1, _BM)
    sel = (jnp.arange(_I, dtype=jnp.int32)[:, None]
           == 2 * jnp.arange(_I // 2, dtype=jnp.int32)[None, :]
           ).astype(jnp.float32)
    grid_spec = pltpu.PrefetchScalarGridSpec(
        num_scalar_prefetch=4,
        grid=(_NBLK,),
        in_specs=[
            pl.BlockSpec((_BM, _H),
                         lambda i, *s: (jnp.minimum(i, s[0][_NBLK] - 1), 0)),
            pl.BlockSpec(memory_space=pltpu.MemorySpace.HBM),
            pl.BlockSpec((1, 1, 2 * _I), lambda i, *s: (s[0][i], 0, 0)),
            pl.BlockSpec((_I, _I // 2), lambda i, *s: (0, 0)),
            pl.BlockSpec(memory_space=pltpu.MemorySpace.HBM),
            pl.BlockSpec((1, 1, _H), lambda i, *s: (s[0][i], 0, 0)),
            pl.BlockSpec((1, 1, _BM), lambda i, *s: (i, 0, 0)),
        ],
        out_specs=pl.BlockSpec(
            (_BM, _H), lambda i, *s: (jnp.minimum(i, s[0][_NBLK] - 1), 0)),
        scratch_shapes=[
            pltpu.VMEM((_NRING, _I, _H), jnp.float32),
            pltpu.VMEM((_NRING, _I, _H), jnp.float32),
            pltpu.VMEM((_NRING, _H // 2, _I), jnp.float32),
            pltpu.VMEM((_NRING, _H // 2, _I), jnp.float32),
            pltpu.SemaphoreType.DMA((_NRING,)),
            pltpu.SemaphoreType.DMA((_NRING,)),
            pltpu.SemaphoreType.DMA((_NRING,)),
            pltpu.SemaphoreType.DMA((_NRING,)),
        ],
    )
    y_pad = pl.pallas_call(
        _ffn_block,
        grid_spec=grid_spec,
        out_shape=jax.ShapeDtypeStruct((_MAXPAD, _H), jnp.float32),
        compiler_params=pltpu.CompilerParams(
            dimension_semantics=("arbitrary",)),
    )(info, runid, fb, runx, x_pad, w1, b1r, sel, w2, b2r, wpr)

    p2 = pos.reshape(_T, _TOPK)
    out2 = _sc_combine(p2[:, 0], p2[:, 1], y_pad)
    return out2.reshape(bsz, seq, hd)
